# baseline (speedup 1.0000x reference)
import functools

import jax
import jax.numpy as jnp
from jax import lax
from jax.experimental import pallas as pl
from jax.experimental.pallas import tpu as pltpu
from jax.experimental.pallas import tpu_sc as plsc

H = 64
NC = 2
NS = 16
NW = NC * NS
GK = 200
SK = 200
BEP = 2000
NPART = 2


def _kr2(W):
    return jnp.kron(jnp.eye(2, dtype=jnp.float32), W)


def _tc_pre(x, Wn, bn, Wi, Wj):
    N = x.shape[0]
    xp = jnp.reshape(x, (N // 2, 2 * x.shape[1]))

    def body(x_ref, wn_ref, bn_ref, wi_ref, wj_ref, h_ref, a_ref, b_ref):
        h = jnp.dot(x_ref[...], wn_ref[...], preferred_element_type=jnp.float32)
        h = h + bn_ref[...]
        h_ref[...] = h
        a_ref[...] = jnp.dot(h, wi_ref[...], preferred_element_type=jnp.float32)
        b_ref[...] = jnp.dot(h, wj_ref[...], preferred_element_type=jnp.float32)

    out = pl.pallas_call(
        body,
        out_shape=[jax.ShapeDtypeStruct((N // 2, 2 * H), jnp.float32)] * 3,
    )(xp, _kr2(Wn), jnp.tile(bn, 2).reshape(1, 2 * H), _kr2(Wi), _kr2(Wj))
    return out


def _sc_gather(A, B, edge_index, off, ne, N):
    epw = ne // NW
    nch = epw // GK
    npt = N // NS
    mesh = plsc.VectorSubcoreMesh(core_axis_name="c", subcore_axis_name="s")

    @functools.partial(
        pl.kernel,
        out_type=jax.ShapeDtypeStruct((ne // 2, 2 * H), jnp.float32),
        mesh=mesh,
        compiler_params=pltpu.CompilerParams(use_tc_tiling_on_sc=False),
        scratch_types=[
            pltpu.VMEM((epw,), jnp.int32),
            pltpu.VMEM((epw,), jnp.int32),
            pltpu.VMEM((2, GK, H), jnp.float32),
            pltpu.VMEM((2, GK, H), jnp.float32),
            pltpu.VMEM((2, GK // 2, 2 * H), jnp.float32),
            pltpu.VMEM_SHARED((N, H), jnp.float32),
            pltpu.SemaphoreType.DMA,
            pltpu.SemaphoreType.DMA,
        ],
    )
    def k(a_hbm, b_hbm, ei_hbm, g_hbm,
          idxd_all, idxs_all, a_v, b_v, o_v, a_sh, sem_a, sem_b):
        cc = lax.axis_index("c")
        ss = lax.axis_index("s")
        wid = ss * NC + cc
        l0 = wid * epw
        pltpu.sync_copy(ei_hbm.at[1, pl.ds(off + l0, epw)], idxd_all)
        pltpu.sync_copy(ei_hbm.at[0, pl.ds(off + l0, epw)], idxs_all)
        pltpu.sync_copy(a_hbm.at[pl.ds(ss * npt, npt)],
                        a_sh.at[pl.ds(ss * npt, npt)])
        plsc.subcore_barrier()
        pltpu.async_copy(a_sh.at[idxd_all.at[pl.ds(0, GK)]], a_v.at[0], sem_a)
        pltpu.async_copy(b_hbm.at[idxs_all.at[pl.ds(0, GK)]], b_v.at[0], sem_b)

        def step(j, carry):
            for p in range(2):
                g = 2 * j + p

                @pl.when(g < nch)
                def _():
                    pltpu.make_async_copy(
                        a_hbm.at[pl.ds(0, GK)], a_v.at[p], sem_a).wait()
                    pltpu.make_async_copy(
                        b_hbm.at[pl.ds(0, GK)], b_v.at[p], sem_b).wait()

                    @pl.when(g + 1 < nch)
                    def _():
                        o = (g + 1) * GK
                        pltpu.async_copy(a_sh.at[idxd_all.at[pl.ds(o, GK)]],
                                         a_v.at[1 - p], sem_a)
                        pltpu.async_copy(b_hbm.at[idxs_all.at[pl.ds(o, GK)]],
                                         b_v.at[1 - p], sem_b)

                    def row(rp, c2):
                        for half in range(2):
                            for t in range(H // 16):
                                sl = pl.ds(t * 16, 16)
                                ol = pl.ds(half * H + t * 16, 16)
                                o_v[p, rp, ol] = (a_v[p, 2 * rp + half, sl]
                                                  + b_v[p, 2 * rp + half, sl])
                        return c2

                    lax.fori_loop(0, GK // 2, row, 0)
                    pltpu.sync_copy(
                        o_v.at[p],
                        g_hbm.at[pl.ds((l0 + g * GK) // 2, GK // 2)])
            return carry

        lax.fori_loop(0, (nch + 1) // 2, step, 0)

    return k(A, B, edge_index)


def _tc_edge(g, e2, W1e, b1, W2, b2, pair_off):
    ne2 = g.shape[0]
    De2 = e2.shape[1]
    blk_off = pair_off // BEP
    W1e2 = jnp.kron(jnp.eye(2, dtype=jnp.float32), W1e)
    W2p = jnp.kron(jnp.eye(2, dtype=jnp.float32), W2)
    b1p = jnp.tile(b1, 2).reshape(1, 2 * H)
    b2p = jnp.tile(b2, 2).reshape(1, 2 * H)

    def body(g_ref, e_ref, w1_ref, b1_ref, w2_ref, b2_ref, m_ref):
        c = jnp.dot(e_ref[...], w1_ref[...], preferred_element_type=jnp.float32)
        p = jnp.maximum(g_ref[...] + c + b1_ref[...], 0.0)
        m = jnp.dot(p, w2_ref[...], preferred_element_type=jnp.float32)
        m_ref[...] = jnp.maximum(m + b2_ref[...], 0.0)

    return pl.pallas_call(
        body,
        grid=(ne2 // BEP,),
        in_specs=[
            pl.BlockSpec((BEP, 2 * H), lambda i: (i, 0)),
            pl.BlockSpec((BEP, De2), lambda i: (i + blk_off, 0)),
            pl.BlockSpec((De2, 2 * H), lambda i: (0, 0)),
            pl.BlockSpec((1, 2 * H), lambda i: (0, 0)),
            pl.BlockSpec((2 * H, 2 * H), lambda i: (0, 0)),
            pl.BlockSpec((1, 2 * H), lambda i: (0, 0)),
        ],
        out_specs=pl.BlockSpec((BEP, 2 * H), lambda i: (i, 0)),
        out_shape=jax.ShapeDtypeStruct((ne2, 2 * H), jnp.float32),
    )(g, e2, W1e2, b1p, W2p, b2p)


def _sc_scatter(M, edge_index, zeros_tile, N, off, ne):
    epw = ne // NW
    nch = epw // SK
    npt = N // NS
    mesh = plsc.VectorSubcoreMesh(core_axis_name="c", subcore_axis_name="s")

    @functools.partial(
        pl.kernel,
        out_type=jax.ShapeDtypeStruct((NC, N, H), jnp.float32),
        mesh=mesh,
        compiler_params=pltpu.CompilerParams(use_tc_tiling_on_sc=False),
        scratch_types=[
            pltpu.VMEM((2, SK), jnp.int32),
            pltpu.VMEM((2, SK // 2, 2 * H), jnp.float32),
            pltpu.VMEM((SK, H), jnp.float32),
            pltpu.VMEM_SHARED((N, H), jnp.float32),
            pltpu.SemaphoreType.DMA,
            pltpu.SemaphoreType.DMA,
        ],
    )
    def k(m_hbm, ei_hbm, z_hbm, out_hbm, idx_v, m_v, m64_v, acc_sh,
          sem_i, sem_m):
        c = lax.axis_index("c")
        s = lax.axis_index("s")
        wid = s * NC + c
        l0 = wid * epw
        pltpu.sync_copy(z_hbm, acc_sh.at[pl.ds(s * npt, npt)])
        plsc.subcore_barrier()
        pltpu.async_copy(ei_hbm.at[1, pl.ds(off + l0, SK)], idx_v.at[0],
                         sem_i)
        pltpu.async_copy(m_hbm.at[pl.ds(l0 // 2, SK // 2)], m_v.at[0], sem_m)

        def step(j, carry):
            for p in range(2):
                g = 2 * j + p

                @pl.when(g < nch)
                def _():
                    pltpu.make_async_copy(
                        ei_hbm.at[1, pl.ds(0, SK)], idx_v.at[p], sem_i).wait()
                    pltpu.make_async_copy(
                        m_hbm.at[pl.ds(0, SK // 2)], m_v.at[p], sem_m).wait()

                    @pl.when(g + 1 < nch)
                    def _():
                        o = l0 + (g + 1) * SK
                        pltpu.async_copy(ei_hbm.at[1, pl.ds(off + o, SK)],
                                         idx_v.at[1 - p], sem_i)
                        pltpu.async_copy(m_hbm.at[pl.ds(o // 2, SK // 2)],
                                         m_v.at[1 - p], sem_m)

                    def row(rp, c2):
                        for half in range(2):
                            for t in range(H // 16):
                                sl = pl.ds(half * H + t * 16, 16)
                                ol = pl.ds(t * 16, 16)
                                m64_v[2 * rp + half, ol] = m_v[p, rp, sl]
                        return c2

                    lax.fori_loop(0, SK // 2, row, 0)
                    pltpu.sync_copy(m64_v, acc_sh.at[idx_v.at[p]], add=True)
            return carry

        lax.fori_loop(0, (nch + 1) // 2, step, 0)
        plsc.subcore_barrier()
        pltpu.sync_copy(acc_sh.at[pl.ds(s * npt, npt)],
                        out_hbm.at[c, pl.ds(s * npt, npt)])

    return k(M, edge_index, zeros_tile)


def _tc_update(h, accs, W1h, W1a, b1, W2, b2, Wi, Wj):
    N2 = h.shape[0]

    def body(h_ref, p0_ref, p1_ref, p2_ref, p3_ref, w1h_ref, w1a_ref, b1_ref,
             w2_ref, b2_ref, wi_ref, wj_ref, h_out, a_out, b_out):
        aggr = ((p0_ref[...] + p1_ref[...]) + (p2_ref[...] + p3_ref[...]))
        u = jnp.dot(h_ref[...], w1h_ref[...], preferred_element_type=jnp.float32)
        u = u + jnp.dot(aggr, w1a_ref[...], preferred_element_type=jnp.float32)
        u = jnp.maximum(u + b1_ref[...], 0.0)
        u = jnp.dot(u, w2_ref[...], preferred_element_type=jnp.float32)
        u = jnp.maximum(u + b2_ref[...], 0.0)
        hn = u + h_ref[...]
        h_out[...] = hn
        a_out[...] = jnp.dot(hn, wi_ref[...], preferred_element_type=jnp.float32)
        b_out[...] = jnp.dot(hn, wj_ref[...], preferred_element_type=jnp.float32)

    pp = [jnp.reshape(a, (NC, N2, 2 * H)) for a in accs]
    return pl.pallas_call(
        body,
        out_shape=[jax.ShapeDtypeStruct((N2, 2 * H), jnp.float32)] * 3,
    )(h, pp[0][0], pp[0][1], pp[1][0], pp[1][1], _kr2(W1h), _kr2(W1a),
      jnp.tile(b1, 2).reshape(1, 2 * H), _kr2(W2),
      jnp.tile(b2, 2).reshape(1, 2 * H), _kr2(Wi), _kr2(Wj))


def _tc_final(h, accs, W1h, W1a, b1, W2, b2, tW1, tb1, tW2c, tb2):
    N2 = h.shape[0]
    Hh = tW1.shape[1]

    def body(h_ref, p0_ref, p1_ref, p2_ref, p3_ref, w1h_ref, w1a_ref, b1_ref,
             w2_ref, b2_ref, tw1_ref, tb1_ref, tw2_ref, tb2_ref,
             h_out, tok_out):
        aggr = ((p0_ref[...] + p1_ref[...]) + (p2_ref[...] + p3_ref[...]))
        u = jnp.dot(h_ref[...], w1h_ref[...], preferred_element_type=jnp.float32)
        u = u + jnp.dot(aggr, w1a_ref[...], preferred_element_type=jnp.float32)
        u = jnp.maximum(u + b1_ref[...], 0.0)
        u = jnp.dot(u, w2_ref[...], preferred_element_type=jnp.float32)
        u = jnp.maximum(u + b2_ref[...], 0.0)
        hn = u + h_ref[...]
        h_out[...] = hn
        t = jnp.dot(hn, tw1_ref[...], preferred_element_type=jnp.float32)
        t = jnp.maximum(t + tb1_ref[...], 0.0)
        tok2 = jnp.dot(t, tw2_ref[...], preferred_element_type=jnp.float32)
        tok_out[...] = tok2 + tb2_ref[...]

    pp = [jnp.reshape(a, (NC, N2, 2 * H)) for a in accs]
    return pl.pallas_call(
        body,
        out_shape=[jax.ShapeDtypeStruct((N2, 2 * H), jnp.float32),
                   jax.ShapeDtypeStruct((N2, 2), jnp.float32)],
    )(h, pp[0][0], pp[0][1], pp[1][0], pp[1][1], _kr2(W1h), _kr2(W1a),
      jnp.tile(b1, 2).reshape(1, 2 * H), _kr2(W2),
      jnp.tile(b2, 2).reshape(1, 2 * H), _kr2(tW1),
      jnp.tile(tb1, 2).reshape(1, 2 * Hh), _kr2(tW2c),
      jnp.full((1, 2), tb2[0], jnp.float32))


def kernel(x, edge_index, edge_attr, params):
    p = params
    N = x.shape[0]
    E = edge_index.shape[1]
    ne = E // NPART
    De = edge_attr.shape[1]
    e2s = [edge_attr[k * ne:(k + 1) * ne].reshape(ne // 2, 2 * De)
           for k in range(NPART)]
    zeros_tile = jnp.zeros((N // NS, H), jnp.float32)

    h, Ap, Bp = _tc_pre(x, p['node_proj_W'], p['node_proj_b'],
                        p['l0_msg_W1'][:H], p['l0_msg_W1'][H:2 * H])
    A = jnp.reshape(Ap, (N, H))
    B = jnp.reshape(Bp, (N, H))
    num_layers = 3
    for i in range(num_layers):
        W1 = p[f'l{i}_msg_W1']
        W1e = p['edge_proj_W'] @ W1[2 * H:]
        b1e = p['edge_proj_b'] @ W1[2 * H:] + p[f'l{i}_msg_b1']
        gs = [None] * NPART
        Ms = [None] * NPART
        accs = [None] * NPART
        for k in range(NPART):
            gs[k] = _sc_gather(A, B, edge_index, k * ne, ne, N)
            if k > 0:
                Ms[k - 1] = _tc_edge(gs[k - 1], e2s[k - 1], W1e, b1e,
                                     p[f'l{i}_msg_W2'], p[f'l{i}_msg_b2'], 0)
        Ms[NPART - 1] = _tc_edge(gs[NPART - 1], e2s[NPART - 1], W1e, b1e,
                                 p[f'l{i}_msg_W2'], p[f'l{i}_msg_b2'], 0)
        for k in range(NPART):
            accs[k] = _sc_scatter(Ms[k], edge_index, zeros_tile, N, k * ne, ne)
        upW1 = p[f'l{i}_up_W1']
        if i < num_layers - 1:
            Wn1 = p[f'l{i + 1}_msg_W1']
            h, Ap, Bp = _tc_update(h, accs, upW1[:H], upW1[H:],
                                   p[f'l{i}_up_b1'], p[f'l{i}_up_W2'],
                                   p[f'l{i}_up_b2'], Wn1[:H], Wn1[H:2 * H])
            A = jnp.reshape(Ap, (N, H))
            B = jnp.reshape(Bp, (N, H))
        else:
            h, tok2 = _tc_final(h, accs, upW1[:H], upW1[H:],
                                p[f'l{i}_up_b1'], p[f'l{i}_up_W2'],
                                p[f'l{i}_up_b2'], p['tok_W1'], p['tok_b1'],
                                p['tok_W2'], p['tok_b2'])
    return jnp.reshape(tok2, (N,)), jnp.reshape(h, (N, H))


# speedup vs baseline: 1.1183x; 1.1183x over previous
import functools

import jax
import jax.numpy as jnp
from jax import lax
from jax.experimental import pallas as pl
from jax.experimental.pallas import tpu as pltpu
from jax.experimental.pallas import tpu_sc as plsc

H = 64
NC = 2
NS = 16
NW = NC * NS
GK = 200
SK = 200
BEP = 2000
NPART = 2


def _kr2(W):
    return jnp.kron(jnp.eye(2, dtype=jnp.float32), W)


def _tc_pre(x, Wn, bn, Wi, Wj):
    N = x.shape[0]
    xp = jnp.reshape(x, (N // 2, 2 * x.shape[1]))

    def body(x_ref, wn_ref, bn_ref, wi_ref, wj_ref, h_ref, a_ref, b_ref):
        h = jnp.dot(x_ref[...], wn_ref[...], preferred_element_type=jnp.float32)
        h = h + bn_ref[...]
        h_ref[...] = h
        a_ref[...] = jnp.dot(h, wi_ref[...], preferred_element_type=jnp.float32)
        b_ref[...] = jnp.dot(h, wj_ref[...], preferred_element_type=jnp.float32)

    out = pl.pallas_call(
        body,
        out_shape=[jax.ShapeDtypeStruct((N // 2, 2 * H), jnp.float32)] * 3,
    )(xp, _kr2(Wn), jnp.tile(bn, 2).reshape(1, 2 * H), _kr2(Wi), _kr2(Wj))
    return out


def _sc_gather(A, B, edge_index, off, ne, N):
    epw = ne // NW
    nch = epw // GK
    npt = N // NS
    mesh = plsc.VectorSubcoreMesh(core_axis_name="c", subcore_axis_name="s")

    @functools.partial(
        pl.kernel,
        out_type=jax.ShapeDtypeStruct((ne // 2, 2 * H), jnp.float32),
        mesh=mesh,
        compiler_params=pltpu.CompilerParams(use_tc_tiling_on_sc=False),
        scratch_types=[
            pltpu.VMEM((epw,), jnp.int32),
            pltpu.VMEM((epw,), jnp.int32),
            pltpu.VMEM((2, GK, H), jnp.float32),
            pltpu.VMEM((2, GK, H), jnp.float32),
            pltpu.VMEM((2, GK // 2, 2 * H), jnp.float32),
            pltpu.VMEM_SHARED((N, H), jnp.float32),
            pltpu.SemaphoreType.DMA,
            pltpu.SemaphoreType.DMA,
        ],
    )
    def k(a_hbm, b_hbm, ei_hbm, g_hbm,
          idxd_all, idxs_all, a_v, b_v, o_v, a_sh, sem_a, sem_b):
        cc = lax.axis_index("c")
        ss = lax.axis_index("s")
        wid = ss * NC + cc
        l0 = wid * epw
        pltpu.sync_copy(ei_hbm.at[1, pl.ds(off + l0, epw)], idxd_all)
        pltpu.sync_copy(ei_hbm.at[0, pl.ds(off + l0, epw)], idxs_all)
        pltpu.sync_copy(a_hbm.at[pl.ds(ss * npt, npt)],
                        a_sh.at[pl.ds(ss * npt, npt)])
        plsc.subcore_barrier()
        pltpu.async_copy(a_sh.at[idxd_all.at[pl.ds(0, GK)]], a_v.at[0], sem_a)
        pltpu.async_copy(b_hbm.at[idxs_all.at[pl.ds(0, GK)]], b_v.at[0], sem_b)

        def step(j, carry):
            for p in range(2):
                g = 2 * j + p

                @pl.when(g < nch)
                def _():
                    pltpu.make_async_copy(
                        a_hbm.at[pl.ds(0, GK)], a_v.at[p], sem_a).wait()
                    pltpu.make_async_copy(
                        b_hbm.at[pl.ds(0, GK)], b_v.at[p], sem_b).wait()

                    @pl.when(g + 1 < nch)
                    def _():
                        o = (g + 1) * GK
                        pltpu.async_copy(a_sh.at[idxd_all.at[pl.ds(o, GK)]],
                                         a_v.at[1 - p], sem_a)
                        pltpu.async_copy(b_hbm.at[idxs_all.at[pl.ds(o, GK)]],
                                         b_v.at[1 - p], sem_b)

                    def row(rp, c2):
                        for half in range(2):
                            for t in range(H // 16):
                                sl = pl.ds(t * 16, 16)
                                ol = pl.ds(half * H + t * 16, 16)
                                o_v[p, rp, ol] = (a_v[p, 2 * rp + half, sl]
                                                  + b_v[p, 2 * rp + half, sl])
                        return c2

                    lax.fori_loop(0, GK // 2, row, 0)
                    pltpu.sync_copy(
                        o_v.at[p],
                        g_hbm.at[pl.ds((l0 + g * GK) // 2, GK // 2)])
            return carry

        lax.fori_loop(0, (nch + 1) // 2, step, 0)

    return k(A, B, edge_index)


def _tc_edge(g, e2, W1e, b1, W2, b2, pair_off):
    ne2 = g.shape[0]
    De2 = e2.shape[1]
    blk_off = pair_off // BEP
    W1e2 = jnp.kron(jnp.eye(2, dtype=jnp.float32), W1e)
    W2p = jnp.kron(jnp.eye(2, dtype=jnp.float32), W2)
    b1p = jnp.tile(b1, 2).reshape(1, 2 * H)
    b2p = jnp.tile(b2, 2).reshape(1, 2 * H)

    def body(g_ref, e_ref, w1_ref, b1_ref, w2_ref, b2_ref, m_ref):
        c = jnp.dot(e_ref[...], w1_ref[...], preferred_element_type=jnp.float32)
        p = jnp.maximum(g_ref[...] + c + b1_ref[...], 0.0)
        m = jnp.dot(p, w2_ref[...], preferred_element_type=jnp.float32)
        m_ref[...] = jnp.maximum(m + b2_ref[...], 0.0)

    return pl.pallas_call(
        body,
        grid=(ne2 // BEP,),
        in_specs=[
            pl.BlockSpec((BEP, 2 * H), lambda i: (i, 0)),
            pl.BlockSpec((BEP, De2), lambda i: (i + blk_off, 0)),
            pl.BlockSpec((De2, 2 * H), lambda i: (0, 0)),
            pl.BlockSpec((1, 2 * H), lambda i: (0, 0)),
            pl.BlockSpec((2 * H, 2 * H), lambda i: (0, 0)),
            pl.BlockSpec((1, 2 * H), lambda i: (0, 0)),
        ],
        out_specs=pl.BlockSpec((BEP, 2 * H), lambda i: (i, 0)),
        out_shape=jax.ShapeDtypeStruct((ne2, 2 * H), jnp.float32),
    )(g, e2, W1e2, b1p, W2p, b2p)


def _sc_scatter(M, edge_index, zeros_tile, N, off, ne):
    epw = ne // NW
    nch = epw // SK
    npt = N // NS
    mesh = plsc.VectorSubcoreMesh(core_axis_name="c", subcore_axis_name="s")

    @functools.partial(
        pl.kernel,
        out_type=jax.ShapeDtypeStruct((NC, N, H), jnp.float32),
        mesh=mesh,
        compiler_params=pltpu.CompilerParams(use_tc_tiling_on_sc=False),
        scratch_types=[
            pltpu.VMEM((2, SK), jnp.int32),
            pltpu.VMEM((2, SK // 2, 2 * H), jnp.float32),
            pltpu.VMEM((SK, H), jnp.float32),
            pltpu.VMEM_SHARED((N, H), jnp.float32),
            pltpu.SemaphoreType.DMA,
            pltpu.SemaphoreType.DMA,
        ],
    )
    def k(m_hbm, ei_hbm, z_hbm, out_hbm, idx_v, m_v, m64_v, acc_sh,
          sem_i, sem_m):
        c = lax.axis_index("c")
        s = lax.axis_index("s")
        wid = s * NC + c
        l0 = wid * epw
        pltpu.sync_copy(z_hbm, acc_sh.at[pl.ds(s * npt, npt)])
        plsc.subcore_barrier()
        pltpu.async_copy(ei_hbm.at[1, pl.ds(off + l0, SK)], idx_v.at[0],
                         sem_i)
        pltpu.async_copy(m_hbm.at[pl.ds(l0 // 2, SK // 2)], m_v.at[0], sem_m)

        def step(j, carry):
            for p in range(2):
                g = 2 * j + p

                @pl.when(g < nch)
                def _():
                    pltpu.make_async_copy(
                        ei_hbm.at[1, pl.ds(0, SK)], idx_v.at[p], sem_i).wait()
                    pltpu.make_async_copy(
                        m_hbm.at[pl.ds(0, SK // 2)], m_v.at[p], sem_m).wait()

                    @pl.when(g + 1 < nch)
                    def _():
                        o = l0 + (g + 1) * SK
                        pltpu.async_copy(ei_hbm.at[1, pl.ds(off + o, SK)],
                                         idx_v.at[1 - p], sem_i)
                        pltpu.async_copy(m_hbm.at[pl.ds(o // 2, SK // 2)],
                                         m_v.at[1 - p], sem_m)

                    def row(rp, c2):
                        for half in range(2):
                            for t in range(H // 16):
                                sl = pl.ds(half * H + t * 16, 16)
                                ol = pl.ds(t * 16, 16)
                                m64_v[2 * rp + half, ol] = m_v[p, rp, sl]
                        return c2

                    lax.fori_loop(0, SK // 2, row, 0)
                    pltpu.sync_copy(m64_v, acc_sh.at[idx_v.at[p]], add=True)
            return carry

        lax.fori_loop(0, (nch + 1) // 2, step, 0)
        plsc.subcore_barrier()
        pltpu.sync_copy(acc_sh.at[pl.ds(s * npt, npt)],
                        out_hbm.at[c, pl.ds(s * npt, npt)])

    return k(M, edge_index, zeros_tile)


def _tc_update(h, accs, W1h, W1a, b1, W2, b2, Wi, Wj):
    N2 = h.shape[0]

    def body(h_ref, p0_ref, p1_ref, p2_ref, p3_ref, w1h_ref, w1a_ref, b1_ref,
             w2_ref, b2_ref, wi_ref, wj_ref, h_out, a_out, b_out):
        aggr = ((p0_ref[...] + p1_ref[...]) + (p2_ref[...] + p3_ref[...]))
        u = jnp.dot(h_ref[...], w1h_ref[...], preferred_element_type=jnp.float32)
        u = u + jnp.dot(aggr, w1a_ref[...], preferred_element_type=jnp.float32)
        u = jnp.maximum(u + b1_ref[...], 0.0)
        u = jnp.dot(u, w2_ref[...], preferred_element_type=jnp.float32)
        u = jnp.maximum(u + b2_ref[...], 0.0)
        hn = u + h_ref[...]
        h_out[...] = hn
        a_out[...] = jnp.dot(hn, wi_ref[...], preferred_element_type=jnp.float32)
        b_out[...] = jnp.dot(hn, wj_ref[...], preferred_element_type=jnp.float32)

    pp = [jnp.reshape(a, (NC, N2, 2 * H)) for a in accs]
    return pl.pallas_call(
        body,
        out_shape=[jax.ShapeDtypeStruct((N2, 2 * H), jnp.float32)] * 3,
    )(h, pp[0][0], pp[0][1], pp[1][0], pp[1][1], _kr2(W1h), _kr2(W1a),
      jnp.tile(b1, 2).reshape(1, 2 * H), _kr2(W2),
      jnp.tile(b2, 2).reshape(1, 2 * H), _kr2(Wi), _kr2(Wj))


def _tc_final(h, accs, W1h, W1a, b1, W2, b2, tW1, tb1, tW2c, tb2):
    N2 = h.shape[0]
    Hh = tW1.shape[1]

    def body(h_ref, p0_ref, p1_ref, p2_ref, p3_ref, w1h_ref, w1a_ref, b1_ref,
             w2_ref, b2_ref, tw1_ref, tb1_ref, tw2_ref, tb2_ref,
             h_out, tok_out):
        aggr = ((p0_ref[...] + p1_ref[...]) + (p2_ref[...] + p3_ref[...]))
        u = jnp.dot(h_ref[...], w1h_ref[...], preferred_element_type=jnp.float32)
        u = u + jnp.dot(aggr, w1a_ref[...], preferred_element_type=jnp.float32)
        u = jnp.maximum(u + b1_ref[...], 0.0)
        u = jnp.dot(u, w2_ref[...], preferred_element_type=jnp.float32)
        u = jnp.maximum(u + b2_ref[...], 0.0)
        hn = u + h_ref[...]
        h_out[...] = hn
        t = jnp.dot(hn, tw1_ref[...], preferred_element_type=jnp.float32)
        t = jnp.maximum(t + tb1_ref[...], 0.0)
        tok2 = jnp.dot(t, tw2_ref[...], preferred_element_type=jnp.float32)
        tok_out[...] = tok2 + tb2_ref[...]

    pp = [jnp.reshape(a, (NC, N2, 2 * H)) for a in accs]
    return pl.pallas_call(
        body,
        out_shape=[jax.ShapeDtypeStruct((N2, 2 * H), jnp.float32),
                   jax.ShapeDtypeStruct((N2, 2), jnp.float32)],
    )(h, pp[0][0], pp[0][1], pp[1][0], pp[1][1], _kr2(W1h), _kr2(W1a),
      jnp.tile(b1, 2).reshape(1, 2 * H), _kr2(W2),
      jnp.tile(b2, 2).reshape(1, 2 * H), _kr2(tW1),
      jnp.tile(tb1, 2).reshape(1, 2 * Hh), _kr2(tW2c),
      jnp.full((1, 2), tb2[0], jnp.float32))


def kernel(x, edge_index, edge_attr, params):
    p = params
    N = x.shape[0]
    E = edge_index.shape[1]
    ne = E // NPART
    e2 = edge_attr.reshape(E // 2, 2 * edge_attr.shape[1])
    zeros_tile = jnp.zeros((N // NS, H), jnp.float32)

    h, Ap, Bp = _tc_pre(x, p['node_proj_W'], p['node_proj_b'],
                        p['l0_msg_W1'][:H], p['l0_msg_W1'][H:2 * H])
    A = jnp.reshape(Ap, (N, H))
    B = jnp.reshape(Bp, (N, H))
    num_layers = 3
    for i in range(num_layers):
        W1 = p[f'l{i}_msg_W1']
        W1e = p['edge_proj_W'] @ W1[2 * H:]
        b1e = p['edge_proj_b'] @ W1[2 * H:] + p[f'l{i}_msg_b1']
        gs = [None] * NPART
        Ms = [None] * NPART
        accs = [None] * NPART
        for k in range(NPART):
            gs[k] = _sc_gather(A, B, edge_index, k * ne, ne, N)
            if k > 0:
                Ms[k - 1] = _tc_edge(gs[k - 1], e2, W1e, b1e,
                                     p[f'l{i}_msg_W2'], p[f'l{i}_msg_b2'],
                                     (k - 1) * ne // 2)
        Ms[NPART - 1] = _tc_edge(gs[NPART - 1], e2, W1e, b1e,
                                 p[f'l{i}_msg_W2'], p[f'l{i}_msg_b2'],
                                 (NPART - 1) * ne // 2)
        for k in range(NPART):
            accs[k] = _sc_scatter(Ms[k], edge_index, zeros_tile, N, k * ne, ne)
        upW1 = p[f'l{i}_up_W1']
        if i < num_layers - 1:
            Wn1 = p[f'l{i + 1}_msg_W1']
            h, Ap, Bp = _tc_update(h, accs, upW1[:H], upW1[H:],
                                   p[f'l{i}_up_b1'], p[f'l{i}_up_W2'],
                                   p[f'l{i}_up_b2'], Wn1[:H], Wn1[H:2 * H])
            A = jnp.reshape(Ap, (N, H))
            B = jnp.reshape(Bp, (N, H))
        else:
            h, tok2 = _tc_final(h, accs, upW1[:H], upW1[H:],
                                p[f'l{i}_up_b1'], p[f'l{i}_up_W2'],
                                p[f'l{i}_up_b2'], p['tok_W1'], p['tok_b1'],
                                p['tok_W2'], p['tok_b2'])
    return jnp.reshape(tok2, (N,)), jnp.reshape(h, (N, H))
